# baseline (speedup 1.0000x reference)
import functools

import jax
import jax.numpy as jnp
import numpy as np
from jax import lax
from jax.experimental import pallas as pl
from jax.experimental.pallas import tpu as pltpu
from jax.experimental.pallas import tpu_sc as plsc

B = 4
N = 2048
RADIUS = 0.15
MAX_DEGREE = 16
INPUT_DIM = 16
HIDDEN_DIM = 256
NDIM = 3
MAX_SUPERNODES = 512
EDGE_TARGET = N * MAX_DEGREE
R2 = RADIUS * RADIUS

NC = 2
NS = 16
NW = NC * NS
ROWS_PER_W = (B * N) // NW


def _freqs():
    epd = HIDDEN_DIM // NDIM
    half = epd // 2
    scale = np.log(10000.0) / (half - 1)
    return np.exp(np.arange(half) * -scale).astype(np.float32)


def _tc1_body(feat_ref, pos_ref, mask_ref, win_ref, bin_ref, wmsg_ref, bmsg_ref,
              a_ref, bm_ref, d2_ref, tau_ref, slot_ref):
    feat = feat_ref[0]
    pos = pos_ref[0]
    mask = mask_ref[0]

    half = (HIDDEN_DIM // NDIM) // 2
    scale = np.float32(np.log(10000.0) / (half - 1))
    fr = jnp.exp(
        lax.broadcasted_iota(jnp.int32, (1, half), 1).astype(jnp.float32)
        * -scale)
    embs = []
    for i in range(NDIM):
        p = pos[:, i:i + 1]
        e = p * fr
        embs.append(jnp.concatenate([jnp.sin(e), jnp.cos(e)], axis=-1))
    emb = jnp.concatenate(embs + [jnp.zeros((N, HIDDEN_DIM - 6 * fr.shape[1]),
                                            jnp.float32)], axis=-1)

    x = jnp.dot(feat, win_ref[...], preferred_element_type=jnp.float32)
    x = x + bin_ref[...] + emb

    a_ref[0] = jnp.dot(x, wmsg_ref[:HIDDEN_DIM, :],
                       preferred_element_type=jnp.float32)
    bm_ref[0] = jnp.dot(x, wmsg_ref[HIDDEN_DIM:, :],
                        preferred_element_type=jnp.float32) + bmsg_ref[...]

    CH = 256
    def d2_chunk(c, _):
        rows = pos_ref[0, pl.ds(c * CH, CH), :]
        acc = jnp.zeros((CH, N), jnp.float32)
        for i in range(NDIM):
            diff = rows[:, i:i + 1] - pos[:, i:i + 1].reshape(1, N)
            acc = acc + diff * diff
        acc = jnp.where(acc <= R2, acc, 2.0)
        d2_ref[0, pl.ds(c * CH, CH), :] = acc
        return 0
    lax.fori_loop(0, N // CH, d2_chunk, 0, unroll=False)

    def count_le(t):
        def ccount(c, acc):
            ch = d2_ref[0, pl.ds(c * CH, CH), :]
            return acc + jnp.sum((ch <= t).astype(jnp.int32))
        return lax.fori_loop(0, N // CH, ccount, jnp.int32(0))

    total_in = count_le(jnp.float32(R2))

    def bs_body(_, carry):
        lo, hi = carry
        mid = 0.5 * (lo + hi)
        ge = count_le(mid) >= EDGE_TARGET
        return jnp.where(ge, lo, mid), jnp.where(ge, mid, hi)

    lo, hi = lax.fori_loop(0, 36, bs_body, (jnp.float32(0.0), jnp.float32(R2)))
    tau = jnp.where(total_in <= EDGE_TARGET, jnp.float32(R2), hi)
    tau_ref[0] = jnp.full((1, 16), tau, jnp.float32)

    cs = mask
    for sh in range(11):
        s = 1 << sh
        cs = cs + jnp.concatenate(
            [jnp.zeros((1, s), jnp.int32), cs[:, :N - s]], axis=1)
    slot = cs - 1
    slot = jnp.where((mask > 0) & (slot < MAX_SUPERNODES), slot, -1)
    slot_ref[0] = slot


def _gelu(t):
    c = np.float32(0.7978845608028654)
    u = c * (t + np.float32(0.044715) * t * t * t)
    e = jnp.exp(2.0 * u)
    th = 1.0 - 2.0 / (e + 1.0)
    return 0.5 * t * (1.0 + th)


def _sc_body(d2_hbm, slot_hbm, tau_hbm, a_hbm, bm_hbm,
             sums_hbm, cnts_hbm,
             slot_v, tau_v, row_v, nbr_v, gath_v, bmi_v, orow_v, cnt_v,
             work_s, sem0, sem1, gsem):
    wid = lax.axis_index("s") * NC + lax.axis_index("c")
    sample = wid // 8
    part = lax.rem(wid, 8)
    base = pl.multiple_of(sample * N + part * ROWS_PER_W, ROWS_PER_W)

    pltpu.sync_copy(slot_hbm.at[pl.ds(base, ROWS_PER_W)], slot_v)
    pltpu.sync_copy(tau_hbm.at[sample], tau_v)

    zero16 = jnp.zeros((16,), jnp.int32)

    def zb(i, _):
        nbr_v[pl.ds(i * 16, 16)] = zero16
        return 0
    lax.fori_loop(0, N // 16, zb, 0)

    work_s[0] = 0

    def wl_group(g, _):
        gb = pl.multiple_of(g * 16, 16)
        sv = slot_v[pl.ds(gb, 16)]
        for l in range(16):
            s = sv[l]

            @pl.when(s >= 0)
            def _add(s=s, r=gb + l):
                w = work_s[0]
                work_s[1 + w] = s * 4096 + r
                work_s[0] = w + 1
        return 0

    lax.fori_loop(0, ROWS_PER_W // 16, wl_group, 0)
    nwork = work_s[0]

    tau = tau_v[...]
    iota16 = lax.broadcasted_iota(jnp.int32, (16,), 0)
    gbase = sample * N
    sems = (sem0, sem1)

    def issue(w, par):
        v = work_s[1 + w]
        grow = base + lax.rem(v, 4096)
        pltpu.async_copy(d2_hbm.at[grow], row_v.at[par], sems[par])
        pltpu.async_copy(bm_hbm.at[grow], bmi_v.at[par], sems[par])

    def do_row(w, par):
        v = work_s[1 + w]
        s = v // 4096
        pltpu.make_async_copy(d2_hbm.at[0], row_v.at[par], sems[par]).wait()
        pltpu.make_async_copy(bm_hbm.at[0], bmi_v.at[par], sems[par]).wait()

        @pl.when(w + 1 < nwork)
        def _prefetch():
            issue(w + 1, 1 - par)

        def cmp_loop(c, off):
            vv = row_v[par, pl.ds(c * 16, 16)]
            m = vv <= tau

            def compact(off):
                cum = plsc.cumsum(m.astype(jnp.int32))
                posn = cum + (off - 1)
                idxv = iota16 + (gbase + c * 16)
                plsc.store_scatter(nbr_v, [posn], idxv, mask=m)
                return off + cum[15]

            return lax.cond(jnp.any(m), compact, lambda off: off, off)

        cnt = lax.fori_loop(0, N // 16, cmp_loop, 0)

        acc0 = tuple(jnp.zeros((16,), jnp.float32) for _ in range(16))

        def gchunk(k, acc):
            ks = pl.multiple_of(k * 32, 32)
            pltpu.async_copy(a_hbm.at[nbr_v.at[pl.ds(ks, 32)]],
                             gath_v, gsem).wait()

            def nacc(n, acc2):
                out = []
                for d in range(16):
                    t = gath_v[n, pl.ds(d * 16, 16)] + bmi_v[par, pl.ds(d * 16, 16)]
                    out.append(acc2[d] + _gelu(t))
                return tuple(out)

            lim = jnp.minimum(32, cnt - k * 32)
            return lax.fori_loop(0, lim, nacc, acc)

        nch = (cnt + 31) // 32
        acc = lax.fori_loop(0, nch, gchunk, acc0)

        for d in range(16):
            orow_v[pl.ds(d * 16, 16)] = acc[d]
        cnt_v[...] = jnp.broadcast_to(cnt.astype(jnp.float32), (16,))

        srow = sample * MAX_SUPERNODES + s
        pltpu.sync_copy(orow_v, sums_hbm.at[srow])
        pltpu.sync_copy(cnt_v, cnts_hbm.at[srow])

    @pl.when(nwork > 0)
    def _prime():
        issue(0, 0)

    def pair_loop(p, _):
        for par in range(2):
            w = 2 * p + par

            @pl.when(w < nwork)
            def _run(w=w, par=par):
                do_row(w, par)
        return 0

    lax.fori_loop(0, (nwork + 1) // 2, pair_loop, 0)


def _tc3_body(sums_ref, cnts_ref, mask_ref, wout_ref, bout_ref, out_ref):
    sums = sums_ref[0]
    cnt = cnts_ref[0][:, 0:1]
    nsn = jnp.sum(mask_ref[0])
    rid = lax.broadcasted_iota(jnp.int32, (MAX_SUPERNODES, 1), 0)
    valid = rid < jnp.minimum(nsn, MAX_SUPERNODES)
    pooled = jnp.where(valid, sums / jnp.maximum(cnt, 1.0), 0.0)
    out_ref[0] = jnp.dot(pooled, wout_ref[...],
                         preferred_element_type=jnp.float32) + bout_ref[...]


def _make_tc1():
    H = HIDDEN_DIM
    return pl.pallas_call(
        _tc1_body,
        grid=(B,),
        in_specs=[
            pl.BlockSpec((1, N, INPUT_DIM), lambda b: (b, 0, 0)),
            pl.BlockSpec((1, N, NDIM), lambda b: (b, 0, 0)),
            pl.BlockSpec((1, 1, N), lambda b: (b, 0, 0)),
            pl.BlockSpec((INPUT_DIM, H), lambda b: (0, 0)),
            pl.BlockSpec((1, H), lambda b: (0, 0)),
            pl.BlockSpec((2 * H, H), lambda b: (0, 0)),
            pl.BlockSpec((1, H), lambda b: (0, 0)),
        ],
        out_specs=[
            pl.BlockSpec((1, N, H), lambda b: (b, 0, 0)),
            pl.BlockSpec((1, N, H), lambda b: (b, 0, 0)),
            pl.BlockSpec((1, N, N), lambda b: (b, 0, 0)),
            pl.BlockSpec((1, 1, 16), lambda b: (b, 0, 0)),
            pl.BlockSpec((1, 1, N), lambda b: (b, 0, 0)),
        ],
        out_shape=[
            jax.ShapeDtypeStruct((B, N, H), jnp.float32),
            jax.ShapeDtypeStruct((B, N, H), jnp.float32),
            jax.ShapeDtypeStruct((B, N, N), jnp.float32),
            jax.ShapeDtypeStruct((B, 1, 16), jnp.float32),
            jax.ShapeDtypeStruct((B, 1, N), jnp.int32),
        ],
    )


def _make_sc():
    H = HIDDEN_DIM
    mesh = plsc.VectorSubcoreMesh(core_axis_name="c", subcore_axis_name="s")
    return pl.kernel(
        _sc_body,
        compiler_params=pltpu.CompilerParams(needs_layout_passes=False),
        out_type=[
            jax.ShapeDtypeStruct((B * MAX_SUPERNODES, H), jnp.float32),
            jax.ShapeDtypeStruct((B * MAX_SUPERNODES, 16), jnp.float32),
        ],
        mesh=mesh,
        scratch_types=[
            pltpu.VMEM((ROWS_PER_W,), jnp.int32),
            pltpu.VMEM((16,), jnp.float32),
            pltpu.VMEM((2, N), jnp.float32),
            pltpu.VMEM((N,), jnp.int32),
            pltpu.VMEM((32, H), jnp.float32),
            pltpu.VMEM((2, H), jnp.float32),
            pltpu.VMEM((H,), jnp.float32),
            pltpu.VMEM((16,), jnp.float32),
            pltpu.SMEM((1 + ROWS_PER_W,), jnp.int32),
            pltpu.SemaphoreType.DMA,
            pltpu.SemaphoreType.DMA,
            pltpu.SemaphoreType.DMA,
        ],
    )


def _make_tc3():
    H = HIDDEN_DIM
    return pl.pallas_call(
        _tc3_body,
        grid=(B,),
        in_specs=[
            pl.BlockSpec((1, MAX_SUPERNODES, H), lambda b: (b, 0, 0)),
            pl.BlockSpec((1, MAX_SUPERNODES, 16), lambda b: (b, 0, 0)),
            pl.BlockSpec((1, 1, N), lambda b: (b, 0, 0)),
            pl.BlockSpec((H, H), lambda b: (0, 0)),
            pl.BlockSpec((1, H), lambda b: (0, 0)),
        ],
        out_specs=pl.BlockSpec((1, MAX_SUPERNODES, H), lambda b: (b, 0, 0)),
        out_shape=jax.ShapeDtypeStruct((B, MAX_SUPERNODES, H), jnp.float32),
    )


def kernel(input_feat, input_pos, supernode_mask, W_in, b_in, W_msg, b_msg,
           W_out, b_out):
    mask_i = supernode_mask.astype(jnp.int32).reshape(B, 1, N)
    a, bm, d2m, tau, slot = _make_tc1()(
        input_feat, input_pos, mask_i, W_in, b_in.reshape(1, HIDDEN_DIM),
        W_msg, b_msg.reshape(1, HIDDEN_DIM))

    sums, cnts = _make_sc()(
        d2m.reshape(B * N, N), slot.reshape(B * N), tau.reshape(B, 16),
        a.reshape(B * N, HIDDEN_DIM), bm.reshape(B * N, HIDDEN_DIM))

    return _make_tc3()(
        sums.reshape(B, MAX_SUPERNODES, HIDDEN_DIM),
        cnts.reshape(B, MAX_SUPERNODES, 16), mask_i, W_out,
        b_out.reshape(1, HIDDEN_DIM))


# speedup vs baseline: 1.0222x; 1.0222x over previous
import functools

import jax
import jax.numpy as jnp
import numpy as np
from jax import lax
from jax.experimental import pallas as pl
from jax.experimental.pallas import tpu as pltpu
from jax.experimental.pallas import tpu_sc as plsc

B = 4
N = 2048
RADIUS = 0.15
MAX_DEGREE = 16
INPUT_DIM = 16
HIDDEN_DIM = 256
NDIM = 3
MAX_SUPERNODES = 512
EDGE_TARGET = N * MAX_DEGREE
R2 = RADIUS * RADIUS

NC = 2
NS = 16
NW = NC * NS
ROWS_PER_W = (B * N) // NW


def _freqs():
    epd = HIDDEN_DIM // NDIM
    half = epd // 2
    scale = np.log(10000.0) / (half - 1)
    return np.exp(np.arange(half) * -scale).astype(np.float32)


def _tc1_body(feat_ref, pos_ref, mask_ref, win_ref, bin_ref, wmsg_ref, bmsg_ref,
              a_ref, bm_ref, d2_ref, tau_ref, slot_ref):
    feat = feat_ref[0]
    pos = pos_ref[0]
    mask = mask_ref[0]

    half = (HIDDEN_DIM // NDIM) // 2
    scale = np.float32(np.log(10000.0) / (half - 1))
    fr = jnp.exp(
        lax.broadcasted_iota(jnp.int32, (1, half), 1).astype(jnp.float32)
        * -scale)
    embs = []
    for i in range(NDIM):
        p = pos[:, i:i + 1]
        e = p * fr
        embs.append(jnp.concatenate([jnp.sin(e), jnp.cos(e)], axis=-1))
    emb = jnp.concatenate(embs + [jnp.zeros((N, HIDDEN_DIM - 6 * fr.shape[1]),
                                            jnp.float32)], axis=-1)

    x = jnp.dot(feat, win_ref[...], preferred_element_type=jnp.float32)
    x = x + bin_ref[...] + emb

    a_ref[0] = jnp.dot(x, wmsg_ref[:HIDDEN_DIM, :],
                       preferred_element_type=jnp.float32)
    bm_ref[0] = jnp.dot(x, wmsg_ref[HIDDEN_DIM:, :],
                        preferred_element_type=jnp.float32) + bmsg_ref[...]

    CH = 256
    def d2_chunk(c, _):
        rows = pos_ref[0, pl.ds(c * CH, CH), :]
        acc = jnp.zeros((CH, N), jnp.float32)
        for i in range(NDIM):
            diff = rows[:, i:i + 1] - pos[:, i:i + 1].reshape(1, N)
            acc = acc + diff * diff
        acc = jnp.where(acc <= R2, acc, 2.0)
        d2_ref[0, pl.ds(c * CH, CH), :] = acc
        return 0
    lax.fori_loop(0, N // CH, d2_chunk, 0, unroll=False)

    def count_le(t):
        def ccount(c, acc):
            ch = d2_ref[0, pl.ds(c * CH, CH), :]
            return acc + jnp.sum((ch <= t).astype(jnp.int32))
        return lax.fori_loop(0, N // CH, ccount, jnp.int32(0))

    total_in = count_le(jnp.float32(R2))

    def bs_body(_, carry):
        lo, hi = carry
        mid = 0.5 * (lo + hi)
        ge = count_le(mid) >= EDGE_TARGET
        return jnp.where(ge, lo, mid), jnp.where(ge, mid, hi)

    lo, hi = lax.fori_loop(0, 36, bs_body, (jnp.float32(0.0), jnp.float32(R2)))
    tau = jnp.where(total_in <= EDGE_TARGET, jnp.float32(R2), hi)
    tau_ref[0] = jnp.full((1, 16), tau, jnp.float32)

    cs = mask
    for sh in range(11):
        s = 1 << sh
        cs = cs + jnp.concatenate(
            [jnp.zeros((1, s), jnp.int32), cs[:, :N - s]], axis=1)
    slot = cs - 1
    slot = jnp.where((mask > 0) & (slot < MAX_SUPERNODES), slot, -1)
    slot_ref[0] = slot


def _gelu(t):
    c = np.float32(0.7978845608028654)
    u = c * (t + np.float32(0.044715) * t * t * t)
    e = jnp.exp(2.0 * u)
    th = 1.0 - 2.0 / (e + 1.0)
    return 0.5 * t * (1.0 + th)


def _sc_body(d2_hbm, slot_hbm, tau_hbm, a_hbm, bm_hbm,
             sums_hbm, cnts_hbm,
             slot_v, tau_v, row_v, nbr_v, gath_v, bmi_v, orow_v, cnt_v,
             work_s, sem0, sem1, gsem):
    wid = lax.axis_index("s") * NC + lax.axis_index("c")
    sample = wid // 8
    part = lax.rem(wid, 8)
    base = pl.multiple_of(sample * N + part * ROWS_PER_W, ROWS_PER_W)

    pltpu.sync_copy(slot_hbm.at[pl.ds(base, ROWS_PER_W)], slot_v)
    pltpu.sync_copy(tau_hbm.at[sample], tau_v)

    zero16 = jnp.zeros((16,), jnp.int32)

    def zb(i, _):
        nbr_v[pl.ds(i * 16, 16)] = zero16
        return 0
    lax.fori_loop(0, N // 16, zb, 0)

    work_s[0] = 0

    def wl_group(g, _):
        gb = pl.multiple_of(g * 16, 16)
        sv = slot_v[pl.ds(gb, 16)]
        for l in range(16):
            s = sv[l]

            @pl.when(s >= 0)
            def _add(s=s, r=gb + l):
                w = work_s[0]
                work_s[1 + w] = s * 4096 + r
                work_s[0] = w + 1
        return 0

    lax.fori_loop(0, ROWS_PER_W // 16, wl_group, 0)
    nwork = work_s[0]

    tau = tau_v[...]
    iota16 = lax.broadcasted_iota(jnp.int32, (16,), 0)
    gbase = sample * N
    sems = (sem0, sem1)

    def issue(w, par):
        v = work_s[1 + w]
        grow = base + lax.rem(v, 4096)
        pltpu.async_copy(d2_hbm.at[grow], row_v.at[par], sems[par])
        pltpu.async_copy(bm_hbm.at[grow], bmi_v.at[par], sems[par])

    def do_row(w, par):
        v = work_s[1 + w]
        s = v // 4096
        pltpu.make_async_copy(d2_hbm.at[0], row_v.at[par], sems[par]).wait()
        pltpu.make_async_copy(bm_hbm.at[0], bmi_v.at[par], sems[par]).wait()

        @pl.when(w + 1 < nwork)
        def _prefetch():
            issue(w + 1, 1 - par)

        def cmp_loop(c, off):
            vv = row_v[par, pl.ds(c * 16, 16)]
            m = vv <= tau
            cum = plsc.cumsum(m.astype(jnp.int32))
            posn = cum + (off - 1)
            idxv = iota16 + (gbase + c * 16)
            plsc.store_scatter(nbr_v, [posn], idxv, mask=m)
            return off + cum[15]

        cnt = lax.fori_loop(0, N // 16, cmp_loop, 0)

        acc0 = tuple(jnp.zeros((16,), jnp.float32) for _ in range(16))

        def gchunk(k, acc):
            ks = pl.multiple_of(k * 32, 32)
            pltpu.async_copy(a_hbm.at[nbr_v.at[pl.ds(ks, 32)]],
                             gath_v, gsem).wait()

            def nacc(n, acc2):
                out = []
                for d in range(16):
                    t = gath_v[n, pl.ds(d * 16, 16)] + bmi_v[par, pl.ds(d * 16, 16)]
                    out.append(acc2[d] + _gelu(t))
                return tuple(out)

            lim = jnp.minimum(32, cnt - k * 32)
            return lax.fori_loop(0, lim, nacc, acc)

        nch = (cnt + 31) // 32
        acc = lax.fori_loop(0, nch, gchunk, acc0)

        for d in range(16):
            orow_v[pl.ds(d * 16, 16)] = acc[d]
        cnt_v[...] = jnp.broadcast_to(cnt.astype(jnp.float32), (16,))

        srow = sample * MAX_SUPERNODES + s
        pltpu.sync_copy(orow_v, sums_hbm.at[srow])
        pltpu.sync_copy(cnt_v, cnts_hbm.at[srow])

    @pl.when(nwork > 0)
    def _prime():
        issue(0, 0)

    def pair_loop(p, _):
        for par in range(2):
            w = 2 * p + par

            @pl.when(w < nwork)
            def _run(w=w, par=par):
                do_row(w, par)
        return 0

    lax.fori_loop(0, (nwork + 1) // 2, pair_loop, 0)


def _tc3_body(sums_ref, cnts_ref, mask_ref, wout_ref, bout_ref, out_ref):
    sums = sums_ref[0]
    cnt = cnts_ref[0][:, 0:1]
    nsn = jnp.sum(mask_ref[0])
    rid = lax.broadcasted_iota(jnp.int32, (MAX_SUPERNODES, 1), 0)
    valid = rid < jnp.minimum(nsn, MAX_SUPERNODES)
    pooled = jnp.where(valid, sums / jnp.maximum(cnt, 1.0), 0.0)
    out_ref[0] = jnp.dot(pooled, wout_ref[...],
                         preferred_element_type=jnp.float32) + bout_ref[...]


def _make_tc1():
    H = HIDDEN_DIM
    return pl.pallas_call(
        _tc1_body,
        grid=(B,),
        in_specs=[
            pl.BlockSpec((1, N, INPUT_DIM), lambda b: (b, 0, 0)),
            pl.BlockSpec((1, N, NDIM), lambda b: (b, 0, 0)),
            pl.BlockSpec((1, 1, N), lambda b: (b, 0, 0)),
            pl.BlockSpec((INPUT_DIM, H), lambda b: (0, 0)),
            pl.BlockSpec((1, H), lambda b: (0, 0)),
            pl.BlockSpec((2 * H, H), lambda b: (0, 0)),
            pl.BlockSpec((1, H), lambda b: (0, 0)),
        ],
        out_specs=[
            pl.BlockSpec((1, N, H), lambda b: (b, 0, 0)),
            pl.BlockSpec((1, N, H), lambda b: (b, 0, 0)),
            pl.BlockSpec((1, N, N), lambda b: (b, 0, 0)),
            pl.BlockSpec((1, 1, 16), lambda b: (b, 0, 0)),
            pl.BlockSpec((1, 1, N), lambda b: (b, 0, 0)),
        ],
        out_shape=[
            jax.ShapeDtypeStruct((B, N, H), jnp.float32),
            jax.ShapeDtypeStruct((B, N, H), jnp.float32),
            jax.ShapeDtypeStruct((B, N, N), jnp.float32),
            jax.ShapeDtypeStruct((B, 1, 16), jnp.float32),
            jax.ShapeDtypeStruct((B, 1, N), jnp.int32),
        ],
    )


def _make_sc():
    H = HIDDEN_DIM
    mesh = plsc.VectorSubcoreMesh(core_axis_name="c", subcore_axis_name="s")
    return pl.kernel(
        _sc_body,
        compiler_params=pltpu.CompilerParams(needs_layout_passes=False),
        out_type=[
            jax.ShapeDtypeStruct((B * MAX_SUPERNODES, H), jnp.float32),
            jax.ShapeDtypeStruct((B * MAX_SUPERNODES, 16), jnp.float32),
        ],
        mesh=mesh,
        scratch_types=[
            pltpu.VMEM((ROWS_PER_W,), jnp.int32),
            pltpu.VMEM((16,), jnp.float32),
            pltpu.VMEM((2, N), jnp.float32),
            pltpu.VMEM((N,), jnp.int32),
            pltpu.VMEM((32, H), jnp.float32),
            pltpu.VMEM((2, H), jnp.float32),
            pltpu.VMEM((H,), jnp.float32),
            pltpu.VMEM((16,), jnp.float32),
            pltpu.SMEM((1 + ROWS_PER_W,), jnp.int32),
            pltpu.SemaphoreType.DMA,
            pltpu.SemaphoreType.DMA,
            pltpu.SemaphoreType.DMA,
        ],
    )


def _make_tc3():
    H = HIDDEN_DIM
    return pl.pallas_call(
        _tc3_body,
        grid=(B,),
        in_specs=[
            pl.BlockSpec((1, MAX_SUPERNODES, H), lambda b: (b, 0, 0)),
            pl.BlockSpec((1, MAX_SUPERNODES, 16), lambda b: (b, 0, 0)),
            pl.BlockSpec((1, 1, N), lambda b: (b, 0, 0)),
            pl.BlockSpec((H, H), lambda b: (0, 0)),
            pl.BlockSpec((1, H), lambda b: (0, 0)),
        ],
        out_specs=pl.BlockSpec((1, MAX_SUPERNODES, H), lambda b: (b, 0, 0)),
        out_shape=jax.ShapeDtypeStruct((B, MAX_SUPERNODES, H), jnp.float32),
    )


def kernel(input_feat, input_pos, supernode_mask, W_in, b_in, W_msg, b_msg,
           W_out, b_out):
    mask_i = supernode_mask.astype(jnp.int32).reshape(B, 1, N)
    a, bm, d2m, tau, slot = _make_tc1()(
        input_feat, input_pos, mask_i, W_in, b_in.reshape(1, HIDDEN_DIM),
        W_msg, b_msg.reshape(1, HIDDEN_DIM))

    sums, cnts = _make_sc()(
        d2m.reshape(B * N, N), slot.reshape(B * N), tau.reshape(B, 16),
        a.reshape(B * N, HIDDEN_DIM), bm.reshape(B * N, HIDDEN_DIM))

    return _make_tc3()(
        sums.reshape(B, MAX_SUPERNODES, HIDDEN_DIM),
        cnts.reshape(B, MAX_SUPERNODES, 16), mask_i, W_out,
        b_out.reshape(1, HIDDEN_DIM))


# speedup vs baseline: 1.1514x; 1.1264x over previous
import functools

import jax
import jax.numpy as jnp
import numpy as np
from jax import lax
from jax.experimental import pallas as pl
from jax.experimental.pallas import tpu as pltpu
from jax.experimental.pallas import tpu_sc as plsc

B = 4
N = 2048
RADIUS = 0.15
MAX_DEGREE = 16
INPUT_DIM = 16
HIDDEN_DIM = 256
NDIM = 3
MAX_SUPERNODES = 512
EDGE_TARGET = N * MAX_DEGREE
R2 = RADIUS * RADIUS

NC = 2
NS = 16
NW = NC * NS
ROWS_PER_W = (B * N) // NW


def _freqs():
    epd = HIDDEN_DIM // NDIM
    half = epd // 2
    scale = np.log(10000.0) / (half - 1)
    return np.exp(np.arange(half) * -scale).astype(np.float32)


def _tc1_body(feat_ref, pos_ref, mask_ref, win_ref, bin_ref, wmsg_ref, bmsg_ref,
              a_ref, bm_ref, d2_ref, tau_ref, slot_ref):
    feat = feat_ref[0]
    pos = pos_ref[0]
    mask = mask_ref[0]

    half = (HIDDEN_DIM // NDIM) // 2
    scale = np.float32(np.log(10000.0) / (half - 1))
    fr = jnp.exp(
        lax.broadcasted_iota(jnp.int32, (1, half), 1).astype(jnp.float32)
        * -scale)
    embs = []
    for i in range(NDIM):
        p = pos[:, i:i + 1]
        e = p * fr
        embs.append(jnp.concatenate([jnp.sin(e), jnp.cos(e)], axis=-1))
    emb = jnp.concatenate(embs + [jnp.zeros((N, HIDDEN_DIM - 6 * fr.shape[1]),
                                            jnp.float32)], axis=-1)

    x = jnp.dot(feat, win_ref[...], preferred_element_type=jnp.float32)
    x = x + bin_ref[...] + emb

    a_ref[0] = jnp.dot(x, wmsg_ref[:HIDDEN_DIM, :],
                       preferred_element_type=jnp.float32)
    bm_ref[0] = jnp.dot(x, wmsg_ref[HIDDEN_DIM:, :],
                        preferred_element_type=jnp.float32) + bmsg_ref[...]

    CH = 256
    def d2_chunk(c, _):
        rows = pos_ref[0, pl.ds(c * CH, CH), :]
        acc = jnp.zeros((CH, N), jnp.float32)
        for i in range(NDIM):
            diff = rows[:, i:i + 1] - pos[:, i:i + 1].reshape(1, N)
            acc = acc + diff * diff
        acc = jnp.where(acc <= R2, acc, 2.0)
        d2_ref[0, pl.ds(c * CH, CH), :] = acc
        return 0
    lax.fori_loop(0, N // CH, d2_chunk, 0, unroll=False)

    def count_le(t):
        def ccount(c, acc):
            ch = d2_ref[0, pl.ds(c * CH, CH), :]
            return acc + jnp.sum((ch <= t).astype(jnp.int32))
        return lax.fori_loop(0, N // CH, ccount, jnp.int32(0))

    total_in = count_le(jnp.float32(R2))

    def bs_body(_, carry):
        lo, hi = carry
        mid = 0.5 * (lo + hi)
        ge = count_le(mid) >= EDGE_TARGET
        return jnp.where(ge, lo, mid), jnp.where(ge, mid, hi)

    lo, hi = lax.fori_loop(0, 36, bs_body, (jnp.float32(0.0), jnp.float32(R2)))
    tau = jnp.where(total_in <= EDGE_TARGET, jnp.float32(R2), hi)
    tau_ref[0] = jnp.full((1, 16), tau, jnp.float32)

    cs = mask
    for sh in range(11):
        s = 1 << sh
        cs = cs + jnp.concatenate(
            [jnp.zeros((1, s), jnp.int32), cs[:, :N - s]], axis=1)
    slot = cs - 1
    slot = jnp.where((mask > 0) & (slot < MAX_SUPERNODES), slot, -1)
    slot_ref[0] = slot


def _gelu(t):
    c = np.float32(0.7978845608028654)
    u = c * (t + np.float32(0.044715) * t * t * t)
    e = jnp.exp(2.0 * u)
    th = 1.0 - 2.0 / (e + 1.0)
    return 0.5 * t * (1.0 + th)


def _sc_body(d2_hbm, slot_hbm, tau_hbm, a_hbm, bm_hbm,
             sums_hbm, cnts_hbm,
             slot_v, tau_v, row_v, nbr_v, gath_v, bmi_v, orow_v, cnt_v,
             work_s, sem0, sem1, gsem):
    wid = lax.axis_index("s") * NC + lax.axis_index("c")
    sample = wid // 8
    part = lax.rem(wid, 8)
    base = pl.multiple_of(sample * N + part * ROWS_PER_W, ROWS_PER_W)

    pltpu.sync_copy(slot_hbm.at[pl.ds(base, ROWS_PER_W)], slot_v)
    pltpu.sync_copy(tau_hbm.at[sample], tau_v)

    zero16 = jnp.zeros((16,), jnp.int32)

    def zb(i, _):
        nbr_v[pl.ds(i * 16, 16)] = zero16
        return 0
    lax.fori_loop(0, N // 16, zb, 0)

    work_s[0] = 0

    def wl_group(g, _):
        gb = pl.multiple_of(g * 16, 16)
        sv = slot_v[pl.ds(gb, 16)]
        for l in range(16):
            s = sv[l]

            @pl.when(s >= 0)
            def _add(s=s, r=gb + l):
                w = work_s[0]
                work_s[1 + w] = s * 4096 + r
                work_s[0] = w + 1
        return 0

    lax.fori_loop(0, ROWS_PER_W // 16, wl_group, 0)
    nwork = work_s[0]

    tau = tau_v[...]
    iota16 = lax.broadcasted_iota(jnp.int32, (16,), 0)
    gbase = sample * N

    def do_row(r, s):
        grow = base + r
        pltpu.async_copy(d2_hbm.at[grow], row_v, sem0)
        pltpu.async_copy(bm_hbm.at[grow], bmi_v, sem1)
        pltpu.make_async_copy(d2_hbm.at[0], row_v, sem0).wait()

        def cmp_loop(c, off):
            vv = row_v[pl.ds(c * 16, 16)]
            m = vv <= tau
            cum = plsc.cumsum(m.astype(jnp.int32))
            posn = cum + (off - 1)
            idxv = iota16 + (gbase + c * 16)
            plsc.store_scatter(nbr_v, [posn], idxv, mask=m)
            return off + cum[15]

        cnt = lax.fori_loop(0, N // 16, cmp_loop, 0)
        pltpu.make_async_copy(bm_hbm.at[0], bmi_v, sem1).wait()

        acc0 = tuple(jnp.zeros((16,), jnp.float32) for _ in range(16))

        def gchunk(k, acc):
            ks = pl.multiple_of(k * 16, 16)
            pltpu.async_copy(a_hbm.at[nbr_v.at[pl.ds(ks, 16)]],
                             gath_v, gsem).wait()

            def nacc(n, acc2):
                out = []
                for d in range(16):
                    t = gath_v[n, pl.ds(d * 16, 16)] + bmi_v[pl.ds(d * 16, 16)]
                    out.append(acc2[d] + _gelu(t))
                return tuple(out)

            lim = jnp.minimum(16, cnt - k * 16)
            return lax.fori_loop(0, lim, nacc, acc)

        nch = (cnt + 15) // 16
        acc = lax.fori_loop(0, nch, gchunk, acc0)

        for d in range(16):
            orow_v[pl.ds(d * 16, 16)] = acc[d]
        cnt_v[...] = jnp.broadcast_to(cnt.astype(jnp.float32), (16,))

        srow = sample * MAX_SUPERNODES + s
        pltpu.sync_copy(orow_v, sums_hbm.at[srow])
        pltpu.sync_copy(cnt_v, cnts_hbm.at[srow])

    def work_loop(w, _):
        v = work_s[1 + w]
        do_row(lax.rem(v, 4096), v // 4096)
        return 0

    lax.fori_loop(0, nwork, work_loop, 0)


def _tc3_body(sums_ref, cnts_ref, mask_ref, wout_ref, bout_ref, out_ref):
    sums = sums_ref[0]
    cnt = cnts_ref[0][:, 0:1]
    nsn = jnp.sum(mask_ref[0])
    rid = lax.broadcasted_iota(jnp.int32, (MAX_SUPERNODES, 1), 0)
    valid = rid < jnp.minimum(nsn, MAX_SUPERNODES)
    pooled = jnp.where(valid, sums / jnp.maximum(cnt, 1.0), 0.0)
    out_ref[0] = jnp.dot(pooled, wout_ref[...],
                         preferred_element_type=jnp.float32) + bout_ref[...]


def _make_tc1():
    H = HIDDEN_DIM
    return pl.pallas_call(
        _tc1_body,
        grid=(B,),
        in_specs=[
            pl.BlockSpec((1, N, INPUT_DIM), lambda b: (b, 0, 0)),
            pl.BlockSpec((1, N, NDIM), lambda b: (b, 0, 0)),
            pl.BlockSpec((1, 1, N), lambda b: (b, 0, 0)),
            pl.BlockSpec((INPUT_DIM, H), lambda b: (0, 0)),
            pl.BlockSpec((1, H), lambda b: (0, 0)),
            pl.BlockSpec((2 * H, H), lambda b: (0, 0)),
            pl.BlockSpec((1, H), lambda b: (0, 0)),
        ],
        out_specs=[
            pl.BlockSpec((1, N, H), lambda b: (b, 0, 0)),
            pl.BlockSpec((1, N, H), lambda b: (b, 0, 0)),
            pl.BlockSpec((1, N, N), lambda b: (b, 0, 0)),
            pl.BlockSpec((1, 1, 16), lambda b: (b, 0, 0)),
            pl.BlockSpec((1, 1, N), lambda b: (b, 0, 0)),
        ],
        out_shape=[
            jax.ShapeDtypeStruct((B, N, H), jnp.float32),
            jax.ShapeDtypeStruct((B, N, H), jnp.float32),
            jax.ShapeDtypeStruct((B, N, N), jnp.float32),
            jax.ShapeDtypeStruct((B, 1, 16), jnp.float32),
            jax.ShapeDtypeStruct((B, 1, N), jnp.int32),
        ],
    )


def _make_sc():
    H = HIDDEN_DIM
    mesh = plsc.VectorSubcoreMesh(core_axis_name="c", subcore_axis_name="s")
    return pl.kernel(
        _sc_body,
        compiler_params=pltpu.CompilerParams(needs_layout_passes=False),
        out_type=[
            jax.ShapeDtypeStruct((B * MAX_SUPERNODES, H), jnp.float32),
            jax.ShapeDtypeStruct((B * MAX_SUPERNODES, 16), jnp.float32),
        ],
        mesh=mesh,
        scratch_types=[
            pltpu.VMEM((ROWS_PER_W,), jnp.int32),
            pltpu.VMEM((16,), jnp.float32),
            pltpu.VMEM((N,), jnp.float32),
            pltpu.VMEM((N,), jnp.int32),
            pltpu.VMEM((16, H), jnp.float32),
            pltpu.VMEM((H,), jnp.float32),
            pltpu.VMEM((H,), jnp.float32),
            pltpu.VMEM((16,), jnp.float32),
            pltpu.SMEM((1 + ROWS_PER_W,), jnp.int32),
            pltpu.SemaphoreType.DMA,
            pltpu.SemaphoreType.DMA,
            pltpu.SemaphoreType.DMA,
        ],
    )


def _make_tc3():
    H = HIDDEN_DIM
    return pl.pallas_call(
        _tc3_body,
        grid=(B,),
        in_specs=[
            pl.BlockSpec((1, MAX_SUPERNODES, H), lambda b: (b, 0, 0)),
            pl.BlockSpec((1, MAX_SUPERNODES, 16), lambda b: (b, 0, 0)),
            pl.BlockSpec((1, 1, N), lambda b: (b, 0, 0)),
            pl.BlockSpec((H, H), lambda b: (0, 0)),
            pl.BlockSpec((1, H), lambda b: (0, 0)),
        ],
        out_specs=pl.BlockSpec((1, MAX_SUPERNODES, H), lambda b: (b, 0, 0)),
        out_shape=jax.ShapeDtypeStruct((B, MAX_SUPERNODES, H), jnp.float32),
    )


def kernel(input_feat, input_pos, supernode_mask, W_in, b_in, W_msg, b_msg,
           W_out, b_out):
    mask_i = supernode_mask.astype(jnp.int32).reshape(B, 1, N)
    a, bm, d2m, tau, slot = _make_tc1()(
        input_feat, input_pos, mask_i, W_in, b_in.reshape(1, HIDDEN_DIM),
        W_msg, b_msg.reshape(1, HIDDEN_DIM))

    sums, cnts = _make_sc()(
        d2m.reshape(B * N, N), slot.reshape(B * N), tau.reshape(B, 16),
        a.reshape(B * N, HIDDEN_DIM), bm.reshape(B * N, HIDDEN_DIM))

    return _make_tc3()(
        sums.reshape(B, MAX_SUPERNODES, HIDDEN_DIM),
        cnts.reshape(B, MAX_SUPERNODES, 16), mask_i, W_out,
        b_out.reshape(1, HIDDEN_DIM))


# speedup vs baseline: 1.2387x; 1.0758x over previous
import functools

import jax
import jax.numpy as jnp
import numpy as np
from jax import lax
from jax.experimental import pallas as pl
from jax.experimental.pallas import tpu as pltpu
from jax.experimental.pallas import tpu_sc as plsc

B = 4
N = 2048
RADIUS = 0.15
MAX_DEGREE = 16
INPUT_DIM = 16
HIDDEN_DIM = 256
NDIM = 3
MAX_SUPERNODES = 512
EDGE_TARGET = N * MAX_DEGREE
R2 = RADIUS * RADIUS

NC = 2
NS = 16
NW = NC * NS
ROWS_PER_W = (B * N) // NW


def _freqs():
    epd = HIDDEN_DIM // NDIM
    half = epd // 2
    scale = np.log(10000.0) / (half - 1)
    return np.exp(np.arange(half) * -scale).astype(np.float32)


def _tc1_body(feat_ref, pos_ref, mask_ref, win_ref, bin_ref, wmsg_ref, bmsg_ref,
              a_ref, bm_ref, d2_ref, tau_ref, slot_ref):
    feat = feat_ref[0]
    pos = pos_ref[0]
    mask = mask_ref[0]

    half = (HIDDEN_DIM // NDIM) // 2
    scale = np.float32(np.log(10000.0) / (half - 1))
    fr = jnp.exp(
        lax.broadcasted_iota(jnp.int32, (1, half), 1).astype(jnp.float32)
        * -scale)
    embs = []
    for i in range(NDIM):
        p = pos[:, i:i + 1]
        e = p * fr
        embs.append(jnp.concatenate([jnp.sin(e), jnp.cos(e)], axis=-1))
    emb = jnp.concatenate(embs + [jnp.zeros((N, HIDDEN_DIM - 6 * fr.shape[1]),
                                            jnp.float32)], axis=-1)

    x = jnp.dot(feat, win_ref[...], preferred_element_type=jnp.float32)
    x = x + bin_ref[...] + emb

    a_ref[0] = jnp.dot(x, wmsg_ref[:HIDDEN_DIM, :],
                       preferred_element_type=jnp.float32)
    bm_ref[0] = jnp.dot(x, wmsg_ref[HIDDEN_DIM:, :],
                        preferred_element_type=jnp.float32) + bmsg_ref[...]

    CH = 256
    def d2_chunk(c, _):
        rows = pos_ref[0, pl.ds(c * CH, CH), :]
        acc = jnp.zeros((CH, N), jnp.float32)
        for i in range(NDIM):
            diff = rows[:, i:i + 1] - pos[:, i:i + 1].reshape(1, N)
            acc = acc + diff * diff
        acc = jnp.where(acc <= R2, acc, 2.0)
        d2_ref[0, pl.ds(c * CH, CH), :] = acc
        return 0
    lax.fori_loop(0, N // CH, d2_chunk, 0, unroll=False)

    def count_le(t):
        def crow(c, acc):
            def cfull(cc, a):
                ch = d2_ref[0, pl.ds(c * CH, CH), pl.ds(cc * CH, CH)]
                return a + jnp.sum((ch <= t).astype(jnp.int32))
            acc = lax.fori_loop(0, c, cfull, acc)
            diag = d2_ref[0, pl.ds(c * CH, CH), pl.ds(c * CH, CH)]
            mlow = (lax.broadcasted_iota(jnp.int32, (CH, CH), 1)
                    < lax.broadcasted_iota(jnp.int32, (CH, CH), 0))
            return acc + jnp.sum(((diag <= t) & mlow).astype(jnp.int32))
        low = lax.fori_loop(0, N // CH, crow, jnp.int32(0))
        return 2 * low + N

    total_in = count_le(jnp.float32(R2))

    def bs_body(_, carry):
        lo, hi = carry
        mid = 0.5 * (lo + hi)
        ge = count_le(mid) >= EDGE_TARGET
        return jnp.where(ge, lo, mid), jnp.where(ge, mid, hi)

    lo, hi = lax.fori_loop(0, 30, bs_body, (jnp.float32(0.0), jnp.float32(R2)))
    tau = jnp.where(total_in <= EDGE_TARGET, jnp.float32(R2), hi)
    tau_ref[0] = jnp.full((1, 16), tau, jnp.float32)

    cs = mask
    for sh in range(11):
        s = 1 << sh
        cs = cs + jnp.concatenate(
            [jnp.zeros((1, s), jnp.int32), cs[:, :N - s]], axis=1)
    slot = cs - 1
    slot = jnp.where((mask > 0) & (slot < MAX_SUPERNODES), slot, -1)
    slot_ref[0] = slot


def _gelu(t):
    c = np.float32(0.7978845608028654)
    u = c * (t + np.float32(0.044715) * t * t * t)
    e = jnp.exp(2.0 * u)
    th = 1.0 - 2.0 / (e + 1.0)
    return 0.5 * t * (1.0 + th)


def _sc_body(d2_hbm, slot_hbm, tau_hbm, a_hbm, bm_hbm,
             sums_hbm, cnts_hbm,
             slot_v, tau_v, row_v, nbr_v, gath_v, bmi_v, orow_v, cnt_v,
             work_s, sem0, sem1, gsem):
    wid = lax.axis_index("s") * NC + lax.axis_index("c")
    sample = wid // 8
    part = lax.rem(wid, 8)
    base = pl.multiple_of(sample * N + part * ROWS_PER_W, ROWS_PER_W)

    pltpu.sync_copy(slot_hbm.at[pl.ds(base, ROWS_PER_W)], slot_v)
    pltpu.sync_copy(tau_hbm.at[sample], tau_v)

    zero16 = jnp.zeros((16,), jnp.int32)

    def zb(i, _):
        nbr_v[pl.ds(i * 16, 16)] = zero16
        return 0
    lax.fori_loop(0, N // 16, zb, 0)

    work_s[0] = 0

    def wl_group(g, _):
        gb = pl.multiple_of(g * 16, 16)
        sv = slot_v[pl.ds(gb, 16)]
        for l in range(16):
            s = sv[l]

            @pl.when(s >= 0)
            def _add(s=s, r=gb + l):
                w = work_s[0]
                work_s[1 + w] = s * 4096 + r
                work_s[0] = w + 1
        return 0

    lax.fori_loop(0, ROWS_PER_W // 16, wl_group, 0)
    nwork = work_s[0]

    tau = tau_v[...]
    iota16 = lax.broadcasted_iota(jnp.int32, (16,), 0)
    gbase = sample * N

    def do_row(r, s):
        grow = base + r
        pltpu.async_copy(d2_hbm.at[grow], row_v, sem0)
        pltpu.async_copy(bm_hbm.at[grow], bmi_v, sem1)
        pltpu.make_async_copy(d2_hbm.at[0], row_v, sem0).wait()

        def cmp_loop(c, off):
            vv = row_v[pl.ds(c * 16, 16)]
            m = vv <= tau
            cum = plsc.cumsum(m.astype(jnp.int32))
            posn = cum + (off - 1)
            idxv = iota16 + (gbase + c * 16)
            plsc.store_scatter(nbr_v, [posn], idxv, mask=m)
            return off + cum[15]

        cnt = lax.fori_loop(0, N // 16, cmp_loop, 0)
        pltpu.make_async_copy(bm_hbm.at[0], bmi_v, sem1).wait()

        acc0 = tuple(jnp.zeros((16,), jnp.float32) for _ in range(16))

        def gchunk(k, acc):
            ks = pl.multiple_of(k * 16, 16)
            pltpu.async_copy(a_hbm.at[nbr_v.at[pl.ds(ks, 16)]],
                             gath_v, gsem).wait()

            def nacc(n, acc2):
                out = []
                for d in range(16):
                    t = gath_v[n, pl.ds(d * 16, 16)] + bmi_v[pl.ds(d * 16, 16)]
                    out.append(acc2[d] + _gelu(t))
                return tuple(out)

            lim = jnp.minimum(16, cnt - k * 16)
            return lax.fori_loop(0, lim, nacc, acc)

        nch = (cnt + 15) // 16
        acc = lax.fori_loop(0, nch, gchunk, acc0)

        for d in range(16):
            orow_v[pl.ds(d * 16, 16)] = acc[d]
        cnt_v[...] = jnp.broadcast_to(cnt.astype(jnp.float32), (16,))

        srow = sample * MAX_SUPERNODES + s
        pltpu.sync_copy(orow_v, sums_hbm.at[srow])
        pltpu.sync_copy(cnt_v, cnts_hbm.at[srow])

    def work_loop(w, _):
        v = work_s[1 + w]
        do_row(lax.rem(v, 4096), v // 4096)
        return 0

    lax.fori_loop(0, nwork, work_loop, 0)


def _tc3_body(sums_ref, cnts_ref, mask_ref, wout_ref, bout_ref, out_ref):
    sums = sums_ref[0]
    cnt = cnts_ref[0][:, 0:1]
    nsn = jnp.sum(mask_ref[0])
    rid = lax.broadcasted_iota(jnp.int32, (MAX_SUPERNODES, 1), 0)
    valid = rid < jnp.minimum(nsn, MAX_SUPERNODES)
    pooled = jnp.where(valid, sums / jnp.maximum(cnt, 1.0), 0.0)
    out_ref[0] = jnp.dot(pooled, wout_ref[...],
                         preferred_element_type=jnp.float32) + bout_ref[...]


def _make_tc1():
    H = HIDDEN_DIM
    return pl.pallas_call(
        _tc1_body,
        grid=(B,),
        in_specs=[
            pl.BlockSpec((1, N, INPUT_DIM), lambda b: (b, 0, 0)),
            pl.BlockSpec((1, N, NDIM), lambda b: (b, 0, 0)),
            pl.BlockSpec((1, 1, N), lambda b: (b, 0, 0)),
            pl.BlockSpec((INPUT_DIM, H), lambda b: (0, 0)),
            pl.BlockSpec((1, H), lambda b: (0, 0)),
            pl.BlockSpec((2 * H, H), lambda b: (0, 0)),
            pl.BlockSpec((1, H), lambda b: (0, 0)),
        ],
        out_specs=[
            pl.BlockSpec((1, N, H), lambda b: (b, 0, 0)),
            pl.BlockSpec((1, N, H), lambda b: (b, 0, 0)),
            pl.BlockSpec((1, N, N), lambda b: (b, 0, 0)),
            pl.BlockSpec((1, 1, 16), lambda b: (b, 0, 0)),
            pl.BlockSpec((1, 1, N), lambda b: (b, 0, 0)),
        ],
        out_shape=[
            jax.ShapeDtypeStruct((B, N, H), jnp.float32),
            jax.ShapeDtypeStruct((B, N, H), jnp.float32),
            jax.ShapeDtypeStruct((B, N, N), jnp.float32),
            jax.ShapeDtypeStruct((B, 1, 16), jnp.float32),
            jax.ShapeDtypeStruct((B, 1, N), jnp.int32),
        ],
    )


def _make_sc():
    H = HIDDEN_DIM
    mesh = plsc.VectorSubcoreMesh(core_axis_name="c", subcore_axis_name="s")
    return pl.kernel(
        _sc_body,
        compiler_params=pltpu.CompilerParams(needs_layout_passes=False),
        out_type=[
            jax.ShapeDtypeStruct((B * MAX_SUPERNODES, H), jnp.float32),
            jax.ShapeDtypeStruct((B * MAX_SUPERNODES, 16), jnp.float32),
        ],
        mesh=mesh,
        scratch_types=[
            pltpu.VMEM((ROWS_PER_W,), jnp.int32),
            pltpu.VMEM((16,), jnp.float32),
            pltpu.VMEM((N,), jnp.float32),
            pltpu.VMEM((N,), jnp.int32),
            pltpu.VMEM((16, H), jnp.float32),
            pltpu.VMEM((H,), jnp.float32),
            pltpu.VMEM((H,), jnp.float32),
            pltpu.VMEM((16,), jnp.float32),
            pltpu.SMEM((1 + ROWS_PER_W,), jnp.int32),
            pltpu.SemaphoreType.DMA,
            pltpu.SemaphoreType.DMA,
            pltpu.SemaphoreType.DMA,
        ],
    )


def _make_tc3():
    H = HIDDEN_DIM
    return pl.pallas_call(
        _tc3_body,
        grid=(B,),
        in_specs=[
            pl.BlockSpec((1, MAX_SUPERNODES, H), lambda b: (b, 0, 0)),
            pl.BlockSpec((1, MAX_SUPERNODES, 16), lambda b: (b, 0, 0)),
            pl.BlockSpec((1, 1, N), lambda b: (b, 0, 0)),
            pl.BlockSpec((H, H), lambda b: (0, 0)),
            pl.BlockSpec((1, H), lambda b: (0, 0)),
        ],
        out_specs=pl.BlockSpec((1, MAX_SUPERNODES, H), lambda b: (b, 0, 0)),
        out_shape=jax.ShapeDtypeStruct((B, MAX_SUPERNODES, H), jnp.float32),
    )


def kernel(input_feat, input_pos, supernode_mask, W_in, b_in, W_msg, b_msg,
           W_out, b_out):
    mask_i = supernode_mask.astype(jnp.int32).reshape(B, 1, N)
    a, bm, d2m, tau, slot = _make_tc1()(
        input_feat, input_pos, mask_i, W_in, b_in.reshape(1, HIDDEN_DIM),
        W_msg, b_msg.reshape(1, HIDDEN_DIM))

    sums, cnts = _make_sc()(
        d2m.reshape(B * N, N), slot.reshape(B * N), tau.reshape(B, 16),
        a.reshape(B * N, HIDDEN_DIM), bm.reshape(B * N, HIDDEN_DIM))

    return _make_tc3()(
        sums.reshape(B, MAX_SUPERNODES, HIDDEN_DIM),
        cnts.reshape(B, MAX_SUPERNODES, 16), mask_i, W_out,
        b_out.reshape(1, HIDDEN_DIM))


# speedup vs baseline: 1.3067x; 1.0549x over previous
import functools

import jax
import jax.numpy as jnp
import numpy as np
from jax import lax
from jax.experimental import pallas as pl
from jax.experimental.pallas import tpu as pltpu
from jax.experimental.pallas import tpu_sc as plsc

B = 4
N = 2048
RADIUS = 0.15
MAX_DEGREE = 16
INPUT_DIM = 16
HIDDEN_DIM = 256
NDIM = 3
MAX_SUPERNODES = 512
EDGE_TARGET = N * MAX_DEGREE
R2 = RADIUS * RADIUS

NC = 2
NS = 16
NW = NC * NS
ROWS_PER_W = (B * N) // NW


def _freqs():
    epd = HIDDEN_DIM // NDIM
    half = epd // 2
    scale = np.log(10000.0) / (half - 1)
    return np.exp(np.arange(half) * -scale).astype(np.float32)


def _tc1_body(feat_ref, pos_ref, mask_ref, win_ref, bin_ref, wmsg_ref, bmsg_ref,
              a_ref, bm_ref, d2_ref, tau_ref, slot_ref):
    feat = feat_ref[0]
    pos = pos_ref[0]
    mask = mask_ref[0]

    half = (HIDDEN_DIM // NDIM) // 2
    scale = np.float32(np.log(10000.0) / (half - 1))
    fr = jnp.exp(
        lax.broadcasted_iota(jnp.int32, (1, half), 1).astype(jnp.float32)
        * -scale)
    embs = []
    for i in range(NDIM):
        p = pos[:, i:i + 1]
        e = p * fr
        embs.append(jnp.concatenate([jnp.sin(e), jnp.cos(e)], axis=-1))
    emb = jnp.concatenate(embs + [jnp.zeros((N, HIDDEN_DIM - 6 * fr.shape[1]),
                                            jnp.float32)], axis=-1)

    x = jnp.dot(feat, win_ref[...], preferred_element_type=jnp.float32)
    x = x + bin_ref[...] + emb

    a_ref[0] = jnp.dot(x, wmsg_ref[:HIDDEN_DIM, :],
                       preferred_element_type=jnp.float32)
    bm_ref[0] = jnp.dot(x, wmsg_ref[HIDDEN_DIM:, :],
                        preferred_element_type=jnp.float32) + bmsg_ref[...]

    CH = 256
    def d2_chunk(c, _):
        rows = pos_ref[0, pl.ds(c * CH, CH), :]
        acc = jnp.zeros((CH, N), jnp.float32)
        for i in range(NDIM):
            diff = rows[:, i:i + 1] - pos[:, i:i + 1].reshape(1, N)
            acc = acc + diff * diff
        acc = jnp.where(acc <= R2, acc, 2.0)
        d2_ref[0, pl.ds(c * CH, CH), :] = acc
        return 0
    lax.fori_loop(0, N // CH, d2_chunk, 0, unroll=False)

    def count_le(t):
        def crow(c, acc):
            def cfull(cc, a):
                ch = d2_ref[0, pl.ds(c * CH, CH), pl.ds(cc * CH, CH)]
                return a + jnp.sum((ch <= t).astype(jnp.int32))
            acc = lax.fori_loop(0, c, cfull, acc)
            diag = d2_ref[0, pl.ds(c * CH, CH), pl.ds(c * CH, CH)]
            mlow = (lax.broadcasted_iota(jnp.int32, (CH, CH), 1)
                    < lax.broadcasted_iota(jnp.int32, (CH, CH), 0))
            return acc + jnp.sum(((diag <= t) & mlow).astype(jnp.int32))
        low = lax.fori_loop(0, N // CH, crow, jnp.int32(0))
        return 2 * low + N

    def bs_body(_, carry):
        lo, hi = carry
        mid = 0.5 * (lo + hi)
        ge = count_le(mid) >= EDGE_TARGET
        return jnp.where(ge, lo, mid), jnp.where(ge, mid, hi)

    lo, hi = lax.fori_loop(0, 30, bs_body, (jnp.float32(0.0), jnp.float32(R2)))
    tau_ref[0] = jnp.full((1, 16), hi, jnp.float32)

    cs = mask
    for sh in range(11):
        s = 1 << sh
        cs = cs + jnp.concatenate(
            [jnp.zeros((1, s), jnp.int32), cs[:, :N - s]], axis=1)
    slot = cs - 1
    slot = jnp.where((mask > 0) & (slot < MAX_SUPERNODES), slot, -1)
    slot_ref[0] = slot


def _gelu(t):
    c = np.float32(0.7978845608028654)
    u = c * (t + np.float32(0.044715) * t * t * t)
    e = jnp.exp(2.0 * u)
    th = 1.0 - 2.0 / (e + 1.0)
    return 0.5 * t * (1.0 + th)


def _sc_body(d2_hbm, slot_hbm, tau_hbm, a_hbm, bm_hbm,
             sums_hbm, cnts_hbm,
             slot_v, tau_v, row_v, nbr_v, gath_v, bmi_v, orow_v, cnt_v,
             work_s, sem0, sem1, gsem):
    wid = lax.axis_index("s") * NC + lax.axis_index("c")
    sample = wid // 8
    part = lax.rem(wid, 8)
    base = pl.multiple_of(sample * N + part * ROWS_PER_W, ROWS_PER_W)

    pltpu.sync_copy(slot_hbm.at[pl.ds(base, ROWS_PER_W)], slot_v)
    pltpu.sync_copy(tau_hbm.at[sample], tau_v)

    zero16 = jnp.zeros((16,), jnp.int32)

    def zb(i, _):
        nbr_v[pl.ds(i * 16, 16)] = zero16
        return 0
    lax.fori_loop(0, N // 16, zb, 0)

    work_s[0] = 0

    def wl_group(g, _):
        gb = pl.multiple_of(g * 16, 16)
        sv = slot_v[pl.ds(gb, 16)]
        for l in range(16):
            s = sv[l]

            @pl.when(s >= 0)
            def _add(s=s, r=gb + l):
                w = work_s[0]
                work_s[1 + w] = s * 4096 + r
                work_s[0] = w + 1
        return 0

    lax.fori_loop(0, ROWS_PER_W // 16, wl_group, 0)
    nwork = work_s[0]

    tau = tau_v[...]
    iota16 = lax.broadcasted_iota(jnp.int32, (16,), 0)
    gbase = sample * N

    def row_of(w):
        return base + lax.rem(work_s[1 + w], 4096)

    def do_row(w, s):
        pltpu.async_copy(bm_hbm.at[row_of(w)], bmi_v, sem1)
        pltpu.make_async_copy(d2_hbm.at[0], row_v, sem0).wait()

        def cmp_loop(c, off):
            vv = row_v[pl.ds(c * 16, 16)]
            m = vv <= tau
            cum = plsc.cumsum(m.astype(jnp.int32))
            posn = cum + (off - 1)
            idxv = iota16 + (gbase + c * 16)
            plsc.store_scatter(nbr_v, [posn], idxv, mask=m)
            return off + cum[15]

        cnt = lax.fori_loop(0, N // 16, cmp_loop, 0)

        @pl.when(w + 1 < nwork)
        def _prefetch():
            pltpu.async_copy(d2_hbm.at[row_of(w + 1)], row_v, sem0)

        pltpu.make_async_copy(bm_hbm.at[0], bmi_v, sem1).wait()

        acc0 = tuple(jnp.zeros((16,), jnp.float32) for _ in range(16))

        def gchunk(k, acc):
            ks = pl.multiple_of(k * 16, 16)
            pltpu.async_copy(a_hbm.at[nbr_v.at[pl.ds(ks, 16)]],
                             gath_v, gsem).wait()

            def nacc(n, acc2):
                out = []
                for d in range(16):
                    t = gath_v[n, pl.ds(d * 16, 16)] + bmi_v[pl.ds(d * 16, 16)]
                    out.append(acc2[d] + _gelu(t))
                return tuple(out)

            lim = jnp.minimum(16, cnt - k * 16)
            return lax.fori_loop(0, lim, nacc, acc)

        nch = (cnt + 15) // 16
        acc = lax.fori_loop(0, nch, gchunk, acc0)

        for d in range(16):
            orow_v[pl.ds(d * 16, 16)] = acc[d]
        cnt_v[...] = jnp.broadcast_to(cnt.astype(jnp.float32), (16,))

        srow = sample * MAX_SUPERNODES + s
        pltpu.sync_copy(orow_v, sums_hbm.at[srow])
        pltpu.sync_copy(cnt_v, cnts_hbm.at[srow])

    @pl.when(nwork > 0)
    def _prime():
        pltpu.async_copy(d2_hbm.at[row_of(0)], row_v, sem0)

    def work_loop(w, _):
        do_row(w, work_s[1 + w] // 4096)
        return 0

    lax.fori_loop(0, nwork, work_loop, 0)


def _tc3_body(sums_ref, cnts_ref, mask_ref, wout_ref, bout_ref, out_ref):
    sums = sums_ref[0]
    cnt = cnts_ref[0][:, 0:1]
    nsn = jnp.sum(mask_ref[0])
    rid = lax.broadcasted_iota(jnp.int32, (MAX_SUPERNODES, 1), 0)
    valid = rid < jnp.minimum(nsn, MAX_SUPERNODES)
    pooled = jnp.where(valid, sums / jnp.maximum(cnt, 1.0), 0.0)
    out_ref[0] = jnp.dot(pooled, wout_ref[...],
                         preferred_element_type=jnp.float32) + bout_ref[...]


def _make_tc1():
    H = HIDDEN_DIM
    return pl.pallas_call(
        _tc1_body,
        grid=(B,),
        in_specs=[
            pl.BlockSpec((1, N, INPUT_DIM), lambda b: (b, 0, 0)),
            pl.BlockSpec((1, N, NDIM), lambda b: (b, 0, 0)),
            pl.BlockSpec((1, 1, N), lambda b: (b, 0, 0)),
            pl.BlockSpec((INPUT_DIM, H), lambda b: (0, 0)),
            pl.BlockSpec((1, H), lambda b: (0, 0)),
            pl.BlockSpec((2 * H, H), lambda b: (0, 0)),
            pl.BlockSpec((1, H), lambda b: (0, 0)),
        ],
        out_specs=[
            pl.BlockSpec((1, N, H), lambda b: (b, 0, 0)),
            pl.BlockSpec((1, N, H), lambda b: (b, 0, 0)),
            pl.BlockSpec((1, N, N), lambda b: (b, 0, 0)),
            pl.BlockSpec((1, 1, 16), lambda b: (b, 0, 0)),
            pl.BlockSpec((1, 1, N), lambda b: (b, 0, 0)),
        ],
        out_shape=[
            jax.ShapeDtypeStruct((B, N, H), jnp.float32),
            jax.ShapeDtypeStruct((B, N, H), jnp.float32),
            jax.ShapeDtypeStruct((B, N, N), jnp.float32),
            jax.ShapeDtypeStruct((B, 1, 16), jnp.float32),
            jax.ShapeDtypeStruct((B, 1, N), jnp.int32),
        ],
    )


def _make_sc():
    H = HIDDEN_DIM
    mesh = plsc.VectorSubcoreMesh(core_axis_name="c", subcore_axis_name="s")
    return pl.kernel(
        _sc_body,
        compiler_params=pltpu.CompilerParams(needs_layout_passes=False),
        out_type=[
            jax.ShapeDtypeStruct((B * MAX_SUPERNODES, H), jnp.float32),
            jax.ShapeDtypeStruct((B * MAX_SUPERNODES, 16), jnp.float32),
        ],
        mesh=mesh,
        scratch_types=[
            pltpu.VMEM((ROWS_PER_W,), jnp.int32),
            pltpu.VMEM((16,), jnp.float32),
            pltpu.VMEM((N,), jnp.float32),
            pltpu.VMEM((N,), jnp.int32),
            pltpu.VMEM((16, H), jnp.float32),
            pltpu.VMEM((H,), jnp.float32),
            pltpu.VMEM((H,), jnp.float32),
            pltpu.VMEM((16,), jnp.float32),
            pltpu.SMEM((1 + ROWS_PER_W,), jnp.int32),
            pltpu.SemaphoreType.DMA,
            pltpu.SemaphoreType.DMA,
            pltpu.SemaphoreType.DMA,
        ],
    )


def _make_tc3():
    H = HIDDEN_DIM
    return pl.pallas_call(
        _tc3_body,
        grid=(B,),
        in_specs=[
            pl.BlockSpec((1, MAX_SUPERNODES, H), lambda b: (b, 0, 0)),
            pl.BlockSpec((1, MAX_SUPERNODES, 16), lambda b: (b, 0, 0)),
            pl.BlockSpec((1, 1, N), lambda b: (b, 0, 0)),
            pl.BlockSpec((H, H), lambda b: (0, 0)),
            pl.BlockSpec((1, H), lambda b: (0, 0)),
        ],
        out_specs=pl.BlockSpec((1, MAX_SUPERNODES, H), lambda b: (b, 0, 0)),
        out_shape=jax.ShapeDtypeStruct((B, MAX_SUPERNODES, H), jnp.float32),
    )


def kernel(input_feat, input_pos, supernode_mask, W_in, b_in, W_msg, b_msg,
           W_out, b_out):
    mask_i = supernode_mask.astype(jnp.int32).reshape(B, 1, N)
    a, bm, d2m, tau, slot = _make_tc1()(
        input_feat, input_pos, mask_i, W_in, b_in.reshape(1, HIDDEN_DIM),
        W_msg, b_msg.reshape(1, HIDDEN_DIM))

    sums, cnts = _make_sc()(
        d2m.reshape(B * N, N), slot.reshape(B * N), tau.reshape(B, 16),
        a.reshape(B * N, HIDDEN_DIM), bm.reshape(B * N, HIDDEN_DIM))

    return _make_tc3()(
        sums.reshape(B, MAX_SUPERNODES, HIDDEN_DIM),
        cnts.reshape(B, MAX_SUPERNODES, 16), mask_i, W_out,
        b_out.reshape(1, HIDDEN_DIM))


# speedup vs baseline: 1.3068x; 1.0001x over previous
import jax
import jax.numpy as jnp
import numpy as np
from jax import lax
from jax.experimental import pallas as pl
from jax.experimental.pallas import tpu as pltpu
from jax.experimental.pallas import tpu_sc as plsc

B = 4
N = 2048
RADIUS = 0.15
MAX_DEGREE = 16
INPUT_DIM = 16
HIDDEN_DIM = 256
NDIM = 3
MAX_SUPERNODES = 512
EDGE_TARGET = N * MAX_DEGREE
R2 = RADIUS * RADIUS

NC = 2
NS = 16
NW = NC * NS
ROWS_PER_W = (B * N) // NW


def _tc1_body(feat_ref, pos_ref, mask_ref, win_ref, bin_ref, wmsg_ref, bmsg_ref,
              a_ref, bm_ref, d2_ref, tau_ref, slot_ref):
    feat = feat_ref[0]
    pos = pos_ref[0]
    mask = mask_ref[0]

    half = (HIDDEN_DIM // NDIM) // 2
    scale = np.float32(np.log(10000.0) / (half - 1))
    fr = jnp.exp(
        lax.broadcasted_iota(jnp.int32, (1, half), 1).astype(jnp.float32)
        * -scale)
    embs = []
    for i in range(NDIM):
        p = pos[:, i:i + 1]
        e = p * fr
        embs.append(jnp.concatenate([jnp.sin(e), jnp.cos(e)], axis=-1))
    emb = jnp.concatenate(embs + [jnp.zeros((N, HIDDEN_DIM - 6 * fr.shape[1]),
                                            jnp.float32)], axis=-1)

    x = jnp.dot(feat, win_ref[...], preferred_element_type=jnp.float32)
    x = x + bin_ref[...] + emb

    a_ref[0] = jnp.dot(x, wmsg_ref[:HIDDEN_DIM, :],
                       preferred_element_type=jnp.float32)
    bm_ref[0] = jnp.dot(x, wmsg_ref[HIDDEN_DIM:, :],
                        preferred_element_type=jnp.float32) + bmsg_ref[...]

    CH = 256
    def d2_chunk(c, _):
        rows = pos_ref[0, pl.ds(c * CH, CH), :]
        acc = jnp.zeros((CH, N), jnp.float32)
        for i in range(NDIM):
            diff = rows[:, i:i + 1] - pos[:, i:i + 1].reshape(1, N)
            acc = acc + diff * diff
        acc = jnp.where(acc <= R2, acc, 2.0)
        d2_ref[0, pl.ds(c * CH, CH), :] = acc
        return 0
    lax.fori_loop(0, N // CH, d2_chunk, 0, unroll=False)

    def count_le(t):
        def crow(c, acc):
            def cfull(cc, a):
                ch = d2_ref[0, pl.ds(c * CH, CH), pl.ds(cc * CH, CH)]
                return a + jnp.sum((ch <= t).astype(jnp.int32))
            acc = lax.fori_loop(0, c, cfull, acc)
            diag = d2_ref[0, pl.ds(c * CH, CH), pl.ds(c * CH, CH)]
            mlow = (lax.broadcasted_iota(jnp.int32, (CH, CH), 1)
                    < lax.broadcasted_iota(jnp.int32, (CH, CH), 0))
            return acc + jnp.sum(((diag <= t) & mlow).astype(jnp.int32))
        low = lax.fori_loop(0, N // CH, crow, jnp.int32(0))
        return 2 * low + N

    def bs_body(_, carry):
        lo, hi = carry
        mid = 0.5 * (lo + hi)
        ge = count_le(mid) >= EDGE_TARGET
        return jnp.where(ge, lo, mid), jnp.where(ge, mid, hi)

    lo, hi = lax.fori_loop(0, 30, bs_body, (jnp.float32(0.0), jnp.float32(R2)))
    tau_ref[0] = jnp.full((1, 16), hi, jnp.float32)

    cs = mask
    for sh in range(11):
        s = 1 << sh
        cs = cs + jnp.concatenate(
            [jnp.zeros((1, s), jnp.int32), cs[:, :N - s]], axis=1)
    slot = cs - 1
    slot = jnp.where((mask > 0) & (slot < MAX_SUPERNODES), slot, -1)
    slot_ref[0] = slot


def _gelu(t):
    c = np.float32(0.7978845608028654)
    u = c * (t + np.float32(0.044715) * t * t * t)
    e = jnp.exp(2.0 * u)
    th = 1.0 - 2.0 / (e + 1.0)
    return 0.5 * t * (1.0 + th)


def _sc_body(d2_hbm, slot_hbm, tau_hbm, a_hbm, bm_hbm,
             sums_hbm, cnts_hbm,
             slot_v, tau_v, row_v, nbr_v, gath_v, bmi_v, orow_v, cnt_v,
             work_s, sem0, sem1, gsem):
    wid = lax.axis_index("s") * NC + lax.axis_index("c")
    sample = wid // 8
    part = lax.rem(wid, 8)
    base = pl.multiple_of(sample * N + part * ROWS_PER_W, ROWS_PER_W)

    pltpu.sync_copy(slot_hbm.at[pl.ds(base, ROWS_PER_W)], slot_v)
    pltpu.sync_copy(tau_hbm.at[sample], tau_v)

    zero16 = jnp.zeros((16,), jnp.int32)

    def zb(i, _):
        nbr_v[pl.ds(i * 16, 16)] = zero16
        return 0
    lax.fori_loop(0, N // 16, zb, 0)

    work_s[0] = 0

    def wl_group(g, _):
        gb = pl.multiple_of(g * 16, 16)
        sv = slot_v[pl.ds(gb, 16)]
        for l in range(16):
            s = sv[l]

            @pl.when(s >= 0)
            def _add(s=s, r=gb + l):
                w = work_s[0]
                work_s[1 + w] = s * 4096 + r
                work_s[0] = w + 1
        return 0

    lax.fori_loop(0, ROWS_PER_W // 16, wl_group, 0)
    nwork = work_s[0]

    tau = tau_v[...]
    iota16 = lax.broadcasted_iota(jnp.int32, (16,), 0)
    gbase = sample * N

    def row_of(w):
        return base + lax.rem(work_s[1 + w], 4096)

    def do_row(w, s):
        pltpu.async_copy(bm_hbm.at[row_of(w)], bmi_v, sem1)
        pltpu.make_async_copy(d2_hbm.at[0], row_v, sem0).wait()

        def cmp_loop(c, off):
            vv = row_v[pl.ds(c * 16, 16)]
            m = vv <= tau
            cum = plsc.cumsum(m.astype(jnp.int32))
            posn = cum + (off - 1)
            idxv = iota16 + (gbase + c * 16)
            plsc.store_scatter(nbr_v, [posn], idxv, mask=m)
            return off + cum[15]

        cnt = lax.fori_loop(0, N // 16, cmp_loop, 0)

        @pl.when(w + 1 < nwork)
        def _prefetch():
            pltpu.async_copy(d2_hbm.at[row_of(w + 1)], row_v, sem0)

        pltpu.make_async_copy(bm_hbm.at[0], bmi_v, sem1).wait()

        acc0 = tuple(jnp.zeros((16,), jnp.float32) for _ in range(16))

        def gchunk(k, acc):
            ks = pl.multiple_of(k * 16, 16)
            pltpu.async_copy(a_hbm.at[nbr_v.at[pl.ds(ks, 16)]],
                             gath_v, gsem).wait()

            def nacc(n, acc2):
                out = []
                for d in range(16):
                    t = gath_v[n, pl.ds(d * 16, 16)] + bmi_v[pl.ds(d * 16, 16)]
                    out.append(acc2[d] + _gelu(t))
                return tuple(out)

            lim = jnp.minimum(16, cnt - k * 16)
            return lax.fori_loop(0, lim, nacc, acc)

        nch = (cnt + 15) // 16
        acc = lax.fori_loop(0, nch, gchunk, acc0)

        for d in range(16):
            orow_v[pl.ds(d * 16, 16)] = acc[d]
        cnt_v[...] = jnp.broadcast_to(cnt.astype(jnp.float32), (16,))

        srow = sample * MAX_SUPERNODES + s
        pltpu.sync_copy(orow_v, sums_hbm.at[srow])
        pltpu.sync_copy(cnt_v, cnts_hbm.at[srow])

    @pl.when(nwork > 0)
    def _prime():
        pltpu.async_copy(d2_hbm.at[row_of(0)], row_v, sem0)

    def work_loop(w, _):
        do_row(w, work_s[1 + w] // 4096)
        return 0

    lax.fori_loop(0, nwork, work_loop, 0)


def _tc3_body(sums_ref, cnts_ref, mask_ref, wout_ref, bout_ref, out_ref):
    sums = sums_ref[0]
    cnt = cnts_ref[0][:, 0:1]
    nsn = jnp.sum(mask_ref[0])
    rid = lax.broadcasted_iota(jnp.int32, (MAX_SUPERNODES, 1), 0)
    valid = rid < jnp.minimum(nsn, MAX_SUPERNODES)
    pooled = jnp.where(valid, sums / jnp.maximum(cnt, 1.0), 0.0)
    out_ref[0] = jnp.dot(pooled, wout_ref[...],
                         preferred_element_type=jnp.float32) + bout_ref[...]


def _make_tc1():
    H = HIDDEN_DIM
    return pl.pallas_call(
        _tc1_body,
        grid=(B,),
        in_specs=[
            pl.BlockSpec((1, N, INPUT_DIM), lambda b: (b, 0, 0)),
            pl.BlockSpec((1, N, NDIM), lambda b: (b, 0, 0)),
            pl.BlockSpec((1, 1, N), lambda b: (b, 0, 0)),
            pl.BlockSpec((INPUT_DIM, H), lambda b: (0, 0)),
            pl.BlockSpec((1, H), lambda b: (0, 0)),
            pl.BlockSpec((2 * H, H), lambda b: (0, 0)),
            pl.BlockSpec((1, H), lambda b: (0, 0)),
        ],
        out_specs=[
            pl.BlockSpec((1, N, H), lambda b: (b, 0, 0)),
            pl.BlockSpec((1, N, H), lambda b: (b, 0, 0)),
            pl.BlockSpec((1, N, N), lambda b: (b, 0, 0)),
            pl.BlockSpec((1, 1, 16), lambda b: (b, 0, 0)),
            pl.BlockSpec((1, 1, N), lambda b: (b, 0, 0)),
        ],
        out_shape=[
            jax.ShapeDtypeStruct((B, N, H), jnp.float32),
            jax.ShapeDtypeStruct((B, N, H), jnp.float32),
            jax.ShapeDtypeStruct((B, N, N), jnp.float32),
            jax.ShapeDtypeStruct((B, 1, 16), jnp.float32),
            jax.ShapeDtypeStruct((B, 1, N), jnp.int32),
        ],
    )


def _make_sc():
    H = HIDDEN_DIM
    mesh = plsc.VectorSubcoreMesh(core_axis_name="c", subcore_axis_name="s")
    return pl.kernel(
        _sc_body,
        compiler_params=pltpu.CompilerParams(needs_layout_passes=False),
        out_type=[
            jax.ShapeDtypeStruct((B * MAX_SUPERNODES, H), jnp.float32),
            jax.ShapeDtypeStruct((B * MAX_SUPERNODES, 16), jnp.float32),
        ],
        mesh=mesh,
        scratch_types=[
            pltpu.VMEM((ROWS_PER_W,), jnp.int32),
            pltpu.VMEM((16,), jnp.float32),
            pltpu.VMEM((N,), jnp.float32),
            pltpu.VMEM((N,), jnp.int32),
            pltpu.VMEM((16, H), jnp.float32),
            pltpu.VMEM((H,), jnp.float32),
            pltpu.VMEM((H,), jnp.float32),
            pltpu.VMEM((16,), jnp.float32),
            pltpu.SMEM((1 + ROWS_PER_W,), jnp.int32),
            pltpu.SemaphoreType.DMA,
            pltpu.SemaphoreType.DMA,
            pltpu.SemaphoreType.DMA,
        ],
    )


def _make_tc3():
    H = HIDDEN_DIM
    return pl.pallas_call(
        _tc3_body,
        grid=(B,),
        in_specs=[
            pl.BlockSpec((1, MAX_SUPERNODES, H), lambda b: (b, 0, 0)),
            pl.BlockSpec((1, MAX_SUPERNODES, 16), lambda b: (b, 0, 0)),
            pl.BlockSpec((1, 1, N), lambda b: (b, 0, 0)),
            pl.BlockSpec((H, H), lambda b: (0, 0)),
            pl.BlockSpec((1, H), lambda b: (0, 0)),
        ],
        out_specs=pl.BlockSpec((1, MAX_SUPERNODES, H), lambda b: (b, 0, 0)),
        out_shape=jax.ShapeDtypeStruct((B, MAX_SUPERNODES, H), jnp.float32),
    )


def kernel(input_feat, input_pos, supernode_mask, W_in, b_in, W_msg, b_msg,
           W_out, b_out):
    mask_i = supernode_mask.astype(jnp.int32).reshape(B, 1, N)
    a, bm, d2m, tau, slot = _make_tc1()(
        input_feat, input_pos, mask_i, W_in, b_in.reshape(1, HIDDEN_DIM),
        W_msg, b_msg.reshape(1, HIDDEN_DIM))

    sums, cnts = _make_sc()(
        d2m.reshape(B * N, N), slot.reshape(B * N), tau.reshape(B, 16),
        a.reshape(B * N, HIDDEN_DIM), bm.reshape(B * N, HIDDEN_DIM))

    return _make_tc3()(
        sums.reshape(B, MAX_SUPERNODES, HIDDEN_DIM),
        cnts.reshape(B, MAX_SUPERNODES, 16), mask_i, W_out,
        b_out.reshape(1, HIDDEN_DIM))
